# baseline (speedup 1.0000x reference)
import jax
import jax.numpy as jnp
from jax.experimental import pallas as pl
from jax.experimental.pallas import tpu as pltpu

_T, _D, _O, _E, _H = 2048, 1024, 1024, 8, 256
_TB = 512


def _moe_body(x_ref, wg_ref, bg_ref, w1_ref, b1_ref, w2_ref, b2_ref,
              out_ref, gl_ref, i1_ref, i2_ref, w1g_ref, w2g_ref):
    e = pl.program_id(1)

    @pl.when(e == 0)
    def _gate():
        xt = x_ref[...]
        logits = jax.lax.dot_general(
            xt, wg_ref[...], (((1,), (1,)), ((), ())),
            preferred_element_type=jnp.float32) + bg_ref[...]
        gl_ref[...] = logits
        ei = jax.lax.broadcasted_iota(jnp.int32, (_TB, _E), 1)
        v1 = jnp.max(logits, axis=1, keepdims=True)
        i1 = jnp.min(jnp.where(logits == v1, ei, _E), axis=1, keepdims=True)
        oh1 = ei == i1
        ml = jnp.where(oh1, -jnp.inf, logits)
        v2 = jnp.max(ml, axis=1, keepdims=True)
        i2 = jnp.min(jnp.where(ml == v2, ei, _E), axis=1, keepdims=True)
        oh2 = ei == i2
        t2 = jnp.exp(v2 - v1)
        g1 = 1.0 / (1.0 + t2)
        g2 = t2 / (1.0 + t2)
        i1_ref[...] = i1
        i2_ref[...] = i2
        w1g_ref[...] = g1
        w2g_ref[...] = g2
        comb = g1 * oh1.astype(jnp.float32) + g2 * oh2.astype(jnp.float32)
        out_ref[...] = jnp.dot(comb, b2_ref[...],
                               preferred_element_type=jnp.float32)

    zero = jnp.zeros((), jnp.float32)
    col = jnp.where(i1_ref[...] == e, w1g_ref[...],
                    jnp.where(i2_ref[...] == e, w2g_ref[...], zero))
    h = jax.lax.dot_general(
        x_ref[...], w1_ref[0], (((1,), (1,)), ((), ())),
        preferred_element_type=jnp.float32)
    h = jnp.maximum(h + b1_ref[0], 0.0)
    hs = h * col
    out_ref[...] += jax.lax.dot_general(
        hs, w2_ref[0], (((1,), (1,)), ((), ())),
        preferred_element_type=jnp.float32)


@jax.jit
def kernel(x, Wg, bg, W1, b1, W2, b2):
    bg2 = bg.reshape(1, _E)
    b1r = b1.reshape(_E, 1, _H)
    out, gl = pl.pallas_call(
        _moe_body,
        grid=(_T // _TB, _E),
        in_specs=[
            pl.BlockSpec((_TB, _D), lambda i, e: (i, 0)),
            pl.BlockSpec((_E, _D), lambda i, e: (0, 0)),
            pl.BlockSpec((1, _E), lambda i, e: (0, 0)),
            pl.BlockSpec((1, _H, _D), lambda i, e: (e, 0, 0)),
            pl.BlockSpec((1, 1, _H), lambda i, e: (e, 0, 0)),
            pl.BlockSpec((1, _O, _H), lambda i, e: (e, 0, 0)),
            pl.BlockSpec((_E, _O), lambda i, e: (0, 0)),
        ],
        out_specs=[
            pl.BlockSpec((_TB, _O), lambda i, e: (i, 0)),
            pl.BlockSpec((_TB, _E), lambda i, e: (i, 0)),
        ],
        out_shape=[
            jax.ShapeDtypeStruct((_T, _O), jnp.float32),
            jax.ShapeDtypeStruct((_T, _E), jnp.float32),
        ],
        scratch_shapes=[
            pltpu.VMEM((_TB, 1), jnp.int32),
            pltpu.VMEM((_TB, 1), jnp.int32),
            pltpu.VMEM((_TB, 1), jnp.float32),
            pltpu.VMEM((_TB, 1), jnp.float32),
        ],
        compiler_params=pltpu.CompilerParams(
            dimension_semantics=("arbitrary", "arbitrary")),
    )(x, Wg, bg2, W1, b1r, W2, b2)
    return (out, gl)


# speedup vs baseline: 1.6660x; 1.6660x over previous
import jax
import jax.numpy as jnp
from jax.experimental import pallas as pl
from jax.experimental.pallas import tpu as pltpu

_T, _D, _O, _E, _H = 2048, 1024, 1024, 8, 256
_TB = 1024


def _moe_body(x_ref, wg_ref, bg_ref, w1_ref, b1_ref, w2_ref, b2_ref,
              out_ref, gl_ref):
    xt = x_ref[...]

    logits = jax.lax.dot_general(
        xt, wg_ref[...], (((1,), (1,)), ((), ())),
        preferred_element_type=jnp.float32) + bg_ref[...]
    gl_ref[...] = logits

    ei = jax.lax.broadcasted_iota(jnp.int32, (_TB, _E), 1)
    v1 = jnp.max(logits, axis=1, keepdims=True)
    i1 = jnp.min(jnp.where(logits == v1, ei, _E), axis=1, keepdims=True)
    oh1 = ei == i1
    ml = jnp.where(oh1, -jnp.inf, logits)
    v2 = jnp.max(ml, axis=1, keepdims=True)
    i2 = jnp.min(jnp.where(ml == v2, ei, _E), axis=1, keepdims=True)
    oh2 = ei == i2
    t2 = jnp.exp(v2 - v1)
    w1g = 1.0 / (1.0 + t2)
    w2g = t2 / (1.0 + t2)
    comb = w1g * oh1.astype(jnp.float32) + w2g * oh2.astype(jnp.float32)

    h = jax.lax.dot_general(
        xt, w1_ref[...], (((1,), (1,)), ((), ())),
        preferred_element_type=jnp.float32)
    h = jnp.maximum(h + b1_ref[...], 0.0)
    eiw = jax.lax.broadcasted_iota(jnp.int32, (_TB, _E * _H), 1) // _H
    zero = jnp.zeros((), jnp.float32)
    combw = jnp.where(eiw == i1, w1g, jnp.where(eiw == i2, w2g, zero))
    hs = h * combw
    acc = jnp.dot(comb, b2_ref[...], preferred_element_type=jnp.float32)
    for e in range(_E):
        acc = acc + jax.lax.dot_general(
            hs[:, e * _H:(e + 1) * _H], w2_ref[e],
            (((1,), (1,)), ((), ())),
            preferred_element_type=jnp.float32)
    out_ref[...] = acc


@jax.jit
def kernel(x, Wg, bg, W1, b1, W2, b2):
    w1c = W1.reshape(_E * _H, _D)
    bg2 = bg.reshape(1, _E)
    b1r = b1.reshape(1, _E * _H)
    out, gl = pl.pallas_call(
        _moe_body,
        grid=(_T // _TB,),
        in_specs=[
            pl.BlockSpec((_TB, _D), lambda i: (i, 0)),
            pl.BlockSpec((_E, _D), lambda i: (0, 0)),
            pl.BlockSpec((1, _E), lambda i: (0, 0)),
            pl.BlockSpec((_E * _H, _D), lambda i: (0, 0)),
            pl.BlockSpec((1, _E * _H), lambda i: (0, 0)),
            pl.BlockSpec((_E, _O, _H), lambda i: (0, 0, 0)),
            pl.BlockSpec((_E, _O), lambda i: (0, 0)),
        ],
        out_specs=[
            pl.BlockSpec((_TB, _O), lambda i: (i, 0)),
            pl.BlockSpec((_TB, _E), lambda i: (i, 0)),
        ],
        out_shape=[
            jax.ShapeDtypeStruct((_T, _O), jnp.float32),
            jax.ShapeDtypeStruct((_T, _E), jnp.float32),
        ],
        compiler_params=pltpu.CompilerParams(
            dimension_semantics=("arbitrary",)),
    )(x, Wg, bg2, w1c, b1r, W2, b2)
    return (out, gl)


# speedup vs baseline: 1.6907x; 1.0148x over previous
import jax
import jax.numpy as jnp
from jax.experimental import pallas as pl
from jax.experimental.pallas import tpu as pltpu

_T, _D, _O, _E, _H = 2048, 1024, 1024, 8, 256
_TB = 512


def _moe_body(x_ref, wg_ref, bg_ref, w1_ref, b1_ref, w2_ref, b2_ref,
              out_ref, gl_ref):
    xt = x_ref[...]

    logits = jax.lax.dot_general(
        xt, wg_ref[...], (((1,), (1,)), ((), ())),
        preferred_element_type=jnp.float32) + bg_ref[...]
    gl_ref[...] = logits

    ei = jax.lax.broadcasted_iota(jnp.int32, (_TB, _E), 1)
    v1 = jnp.max(logits, axis=1, keepdims=True)
    i1 = jnp.min(jnp.where(logits == v1, ei, _E), axis=1, keepdims=True)
    oh1 = ei == i1
    ml = jnp.where(oh1, -jnp.inf, logits)
    v2 = jnp.max(ml, axis=1, keepdims=True)
    i2 = jnp.min(jnp.where(ml == v2, ei, _E), axis=1, keepdims=True)
    oh2 = ei == i2
    t2 = jnp.exp(v2 - v1)
    w1g = 1.0 / (1.0 + t2)
    w2g = t2 / (1.0 + t2)
    comb = w1g * oh1.astype(jnp.float32) + w2g * oh2.astype(jnp.float32)

    h = jax.lax.dot_general(
        xt, w1_ref[...], (((1,), (1,)), ((), ())),
        preferred_element_type=jnp.float32)
    h = jnp.maximum(h + b1_ref[...], 0.0)
    eiw = jax.lax.broadcasted_iota(jnp.int32, (_TB, _E * _H), 1) // _H
    zero = jnp.zeros((), jnp.float32)
    combw = jnp.where(eiw == i1, w1g, jnp.where(eiw == i2, w2g, zero))
    hs = h * combw
    acc = jnp.dot(comb, b2_ref[...], preferred_element_type=jnp.float32)
    for e in range(_E):
        acc = acc + jax.lax.dot_general(
            hs[:, e * _H:(e + 1) * _H], w2_ref[e],
            (((1,), (1,)), ((), ())),
            preferred_element_type=jnp.float32)
    out_ref[...] = acc


@jax.jit
def kernel(x, Wg, bg, W1, b1, W2, b2):
    w1c = W1.reshape(_E * _H, _D)
    bg2 = bg.reshape(1, _E)
    b1r = b1.reshape(1, _E * _H)
    out, gl = pl.pallas_call(
        _moe_body,
        grid=(_T // _TB,),
        in_specs=[
            pl.BlockSpec((_TB, _D), lambda i: (i, 0)),
            pl.BlockSpec((_E, _D), lambda i: (0, 0)),
            pl.BlockSpec((1, _E), lambda i: (0, 0)),
            pl.BlockSpec((_E * _H, _D), lambda i: (0, 0)),
            pl.BlockSpec((1, _E * _H), lambda i: (0, 0)),
            pl.BlockSpec((_E, _O, _H), lambda i: (0, 0, 0)),
            pl.BlockSpec((_E, _O), lambda i: (0, 0)),
        ],
        out_specs=[
            pl.BlockSpec((_TB, _O), lambda i: (i, 0)),
            pl.BlockSpec((_TB, _E), lambda i: (i, 0)),
        ],
        out_shape=[
            jax.ShapeDtypeStruct((_T, _O), jnp.float32),
            jax.ShapeDtypeStruct((_T, _E), jnp.float32),
        ],
        compiler_params=pltpu.CompilerParams(
            dimension_semantics=("arbitrary",)),
    )(x, Wg, bg2, w1c, b1r, W2, b2)
    return (out, gl)


# speedup vs baseline: 1.8928x; 1.1195x over previous
import jax
import jax.numpy as jnp
from jax.experimental import pallas as pl
from jax.experimental.pallas import tpu as pltpu

_T, _D, _O, _E, _H = 2048, 1024, 1024, 8, 256
_TB = 512


def _moe_body(x_ref, wg_ref, bg_ref, w1_ref, b1_ref, w2_ref, b2_ref,
              out_ref, gl_ref):
    xt = x_ref[...]

    logits = jax.lax.dot_general(
        xt.astype(jnp.bfloat16), wg_ref[...].astype(jnp.bfloat16),
        (((1,), (1,)), ((), ())),
        preferred_element_type=jnp.float32) + bg_ref[...]
    gl_ref[...] = logits

    ei = jax.lax.broadcasted_iota(jnp.int32, (_TB, _E), 1)
    v1 = jnp.max(logits, axis=1, keepdims=True)
    i1 = jnp.min(jnp.where(logits == v1, ei, _E), axis=1, keepdims=True)
    oh1 = ei == i1
    ml = jnp.where(oh1, -jnp.inf, logits)
    v2 = jnp.max(ml, axis=1, keepdims=True)
    i2 = jnp.min(jnp.where(ml == v2, ei, _E), axis=1, keepdims=True)
    oh2 = ei == i2
    t2 = jnp.exp(v2 - v1)
    w1g = 1.0 / (1.0 + t2)
    w2g = t2 / (1.0 + t2)
    comb = w1g * oh1.astype(jnp.float32) + w2g * oh2.astype(jnp.float32)

    xb = xt.astype(jnp.bfloat16)
    h = jax.lax.dot_general(
        xb, w1_ref[...].astype(jnp.bfloat16), (((1,), (1,)), ((), ())),
        preferred_element_type=jnp.float32)
    h = jnp.maximum(h + b1_ref[...], 0.0)
    eiw = jax.lax.broadcasted_iota(jnp.int32, (_TB, _E * _H), 1) // _H
    zero = jnp.zeros((), jnp.float32)
    combw = jnp.where(eiw == i1, w1g, jnp.where(eiw == i2, w2g, zero))
    hs = h * combw
    acc = jnp.dot(comb, b2_ref[...], preferred_element_type=jnp.float32)
    for e in range(_E):
        acc = acc + jax.lax.dot_general(
            hs[:, e * _H:(e + 1) * _H].astype(jnp.bfloat16),
            w2_ref[e].astype(jnp.bfloat16),
            (((1,), (1,)), ((), ())),
            preferred_element_type=jnp.float32)
    out_ref[...] = acc


@jax.jit
def kernel(x, Wg, bg, W1, b1, W2, b2):
    w1c = W1.reshape(_E * _H, _D)
    bg2 = bg.reshape(1, _E)
    b1r = b1.reshape(1, _E * _H)
    out, gl = pl.pallas_call(
        _moe_body,
        grid=(_T // _TB,),
        in_specs=[
            pl.BlockSpec((_TB, _D), lambda i: (i, 0)),
            pl.BlockSpec((_E, _D), lambda i: (0, 0)),
            pl.BlockSpec((1, _E), lambda i: (0, 0)),
            pl.BlockSpec((_E * _H, _D), lambda i: (0, 0)),
            pl.BlockSpec((1, _E * _H), lambda i: (0, 0)),
            pl.BlockSpec((_E, _O, _H), lambda i: (0, 0, 0)),
            pl.BlockSpec((_E, _O), lambda i: (0, 0)),
        ],
        out_specs=[
            pl.BlockSpec((_TB, _O), lambda i: (i, 0)),
            pl.BlockSpec((_TB, _E), lambda i: (i, 0)),
        ],
        out_shape=[
            jax.ShapeDtypeStruct((_T, _O), jnp.float32),
            jax.ShapeDtypeStruct((_T, _E), jnp.float32),
        ],
        compiler_params=pltpu.CompilerParams(
            dimension_semantics=("arbitrary",)),
    )(x, Wg, bg2, w1c, b1r, W2, b2)
    return (out, gl)
